# baseline (speedup 1.0000x reference)
import jax
import jax.numpy as jnp
from jax import lax
from jax.experimental import pallas as pl
from jax.experimental.pallas import tpu as pltpu
from jax.experimental.pallas import tpu_sc as plsc

D_MODEL = 128
NC = 2
NS = 16
NW = NC * NS

G = 128
NCHUNK = 200
NB = 4
AHEAD = NB - 1
RB = 2
SLAB = NS * G


def _emb_body(x_hbm, table_hbm, out_hbm, idx_v, shr, *scratch):
    rows = scratch[:NB]
    wsem = scratch[NB:2 * NB]
    dsem = scratch[2 * NB:2 * NB + RB]
    gsem = scratch[2 * NB + RB:]
    cc = lax.axis_index("c")
    ss = lax.axis_index("s")
    pltpu.sync_copy(x_hbm.at[cc, ss], idx_v)

    def t0(fn):
        pl.when(ss == 0)(fn)

    def fire_g(ci, b):
        pltpu.async_copy(table_hbm.at[idx_v.at[ci]], rows[b], gsem[b])

    def wait_g(b):
        pltpu.make_async_copy(table_hbm.at[pl.ds(0, G)], rows[b], gsem[b]).wait()

    def fire_x(b, r):
        pltpu.async_copy(rows[b], shr.at[r, pl.ds(ss * G, G)], wsem[b])

    def wait_x(b):
        pltpu.make_async_copy(rows[b], shr.at[0, pl.ds(0, G)], wsem[b]).wait()

    def fire_d(ci, r):
        pltpu.async_copy(
            shr.at[r], out_hbm.at[pl.ds((ci * NC + cc) * SLAB, SLAB)], dsem[r])

    def wait_d(r):
        pltpu.make_async_copy(
            shr.at[r], out_hbm.at[pl.ds(0, SLAB)], dsem[r]).wait()

    for b in range(AHEAD):
        fire_g(b, b)

    def slot(ci, b, first=False, may_fire=True):
        r = b % RB
        wait_g(b)
        if not (first and b == 0):
            wait_x((b + NB - 1) % NB)
        if not (first and b < RB):
            t0(lambda: wait_d(r))
        plsc.subcore_barrier()
        if not (first and b == 0):
            t0(lambda: fire_d(ci - 1, (r + RB - 1) % RB))
        fire_x(b, r)
        if may_fire:
            fire_g(ci + AHEAD, (b + NB - 1) % NB)

    for b in range(NB):
        slot(b, b, first=True)

    def step(s, carry):
        for b in range(NB):
            slot(s * NB + b, b)
        return carry

    lax.fori_loop(1, NCHUNK // NB - 1, step, 0)

    s = NCHUNK // NB - 1
    for b in range(NB):
        ci = s * NB + b
        slot(ci, b, may_fire=(ci + AHEAD < NCHUNK))

    wait_x((NCHUNK - 1) % NB)
    plsc.subcore_barrier()
    t0(lambda: fire_d(NCHUNK - 1, (NCHUNK - 1) % RB))
    t0(lambda: wait_d((NCHUNK - 2) % RB))
    t0(lambda: wait_d((NCHUNK - 1) % RB))


@jax.jit
def _emb(xf, table):
    mesh = plsc.VectorSubcoreMesh(core_axis_name="c", subcore_axis_name="s")
    kern = pl.kernel(
        _emb_body,
        out_type=jax.ShapeDtypeStruct((NCHUNK * NC * SLAB, D_MODEL), jnp.float32),
        mesh=mesh,
        scratch_types=(
            [pltpu.VMEM((NCHUNK, G), jnp.int32),
             pltpu.VMEM_SHARED((RB, SLAB, D_MODEL), jnp.float32)]
            + [pltpu.VMEM((G, D_MODEL), jnp.float32) for _ in range(NB)]
            + [pltpu.SemaphoreType.DMA for _ in range(NB + RB + NB)]
        ),
    )
    return kern(xf, table)


def kernel(x, table):
    b, t = x.shape
    xf = (x.reshape(NCHUNK, NC, NS, G).transpose(1, 2, 0, 3)
          .astype(jnp.int32))
    out = _emb(xf, table)
    return out.reshape(b, t, D_MODEL)


# speedup vs baseline: 1.2555x; 1.2555x over previous
import jax
import jax.numpy as jnp
from jax import lax
from jax.experimental import pallas as pl
from jax.experimental.pallas import tpu as pltpu
from jax.experimental.pallas import tpu_sc as plsc

D_MODEL = 128
NC = 2
NS = 16
NW = NC * NS

G = 128
NCHUNK = 200
NB = 4
AHEAD = NB - 1
SPH = 1
RB = 2
NGROUP = NCHUNK // SPH


def _emb_body(x_hbm, table_hbm, out_hbm, idx_v, shr, *scratch):
    rows = scratch[:NB]
    wsem = scratch[NB:2 * NB]
    dsem = scratch[2 * NB:2 * NB + RB]
    gsem = scratch[2 * NB + RB:]
    cc = lax.axis_index("c")
    ss = lax.axis_index("s")
    wid = ss * NC + cc
    pltpu.sync_copy(x_hbm.at[wid], idx_v)
    base = wid * (NCHUNK * G)

    def fire_g(ci, b):
        pltpu.async_copy(table_hbm.at[idx_v.at[ci]], rows[b], gsem[b])

    def wait_g(b):
        pltpu.make_async_copy(table_hbm.at[pl.ds(0, G)], rows[b], gsem[b]).wait()

    def fire_x(b, r):
        pltpu.async_copy(rows[b], shr.at[ss, r], wsem[b])

    def wait_x(b):
        pltpu.make_async_copy(rows[b], shr.at[ss, 0], wsem[b]).wait()

    def fire_d(gi, r):
        pltpu.async_copy(
            shr.at[ss, r], out_hbm.at[pl.ds(base + gi * SPH * G, SPH * G)], dsem[r])

    def wait_d(r):
        pltpu.make_async_copy(
            shr.at[ss, r], out_hbm.at[pl.ds(base, SPH * G)], dsem[r]).wait()

    for b in range(AHEAD):
        fire_g(b, b)

    def make_step(first=False, last=False):
        def step(s, carry):
            for b in range(NB):
                ci = s * NB + b
                r = b % RB
                wait_g(b)
                if not (first and b == 0):
                    wait_x((b + NB - 1) % NB)
                    pltpu.async_copy(
                        shr.at[ss, (r + RB - 1) % RB],
                        out_hbm.at[pl.ds(base + (ci - 1) * G, G)],
                        dsem[(r + RB - 1) % RB])
                if (not first) or b >= RB:
                    wait_d(r)
                fire_x(b, r)
                if (not last) or (ci + AHEAD < NCHUNK):
                    fire_g(ci + AHEAD, (b + NB - 1) % NB)
            return carry
        return step

    make_step(first=True)(0, 0)
    lax.fori_loop(1, NCHUNK // NB - 1, make_step(), 0)
    make_step(last=True)(NCHUNK // NB - 1, 0)

    wait_x((NCHUNK - 1) % NB)
    fire_d(NGROUP - 1, (NGROUP - 1) % RB)
    wait_d((NGROUP - 2) % RB)
    wait_d((NGROUP - 1) % RB)


@jax.jit
def _emb(xf, table):
    mesh = plsc.VectorSubcoreMesh(core_axis_name="c", subcore_axis_name="s")
    kern = pl.kernel(
        _emb_body,
        out_type=jax.ShapeDtypeStruct((NW * NCHUNK * G, D_MODEL), jnp.float32),
        mesh=mesh,
        scratch_types=(
            [pltpu.VMEM((NCHUNK, G), jnp.int32),
             pltpu.VMEM_SHARED((NS, RB, SPH * G, D_MODEL), jnp.float32)]
            + [pltpu.VMEM((G, D_MODEL), jnp.float32) for _ in range(NB)]
            + [pltpu.SemaphoreType.DMA for _ in range(NB + RB + NB)]
        ),
    )
    return kern(xf, table)


def kernel(x, table):
    b, t = x.shape
    xf = x.reshape(NW, NCHUNK, G).astype(jnp.int32)
    out = _emb(xf, table)
    return out.reshape(b, t, D_MODEL)
